# baseline (speedup 1.0000x reference)
import functools

import jax
import jax.numpy as jnp
from jax import lax
from jax.experimental import pallas as pl
from jax.experimental.pallas import tpu as pltpu
from jax.experimental.pallas import tpu_sc as plsc

N = 8192
D = 64
K = 1024
BLK = 1024
HALF_B = 4
HGRID = HALF_B
NH = HALF_B * 1024

_LOSS_W = 1.25
_ENTROPY_W = 0.1


def _tc_body(is_final, x_ref, cb_ref, cm_in, zc_in, rs_in,
             tok_ref, cm_out, zc_out, rs_out,
             cbt_s, c2_s, colmin_s, zcol_s, iota_s, rowsum_s):
    i = pl.program_id(0)

    @pl.when(i == 0)
    def _init():
        cbt = cb_ref[...].T
        cbt_s[...] = cbt.astype(jnp.bfloat16)
        c2_s[...] = jnp.sum(cbt * cbt, axis=0, keepdims=True)
        colmin_s[...] = cm_in[...]
        zcol_s[...] = zc_in[...]
        iota_s[...] = lax.broadcasted_iota(jnp.int32, (1, K), 1).astype(jnp.float32)
        rowsum_s[0, 0] = rs_in[0, 0]

    x = x_ref[...].reshape(BLK, D)
    v2 = jnp.sum(x * x, axis=1, keepdims=True)
    cross2 = jnp.dot((x + x).astype(jnp.bfloat16), cbt_s[...],
                     preferred_element_type=jnp.float32)
    dist = (v2 - cross2) + c2_s[...]

    rowmin = jnp.min(dist, axis=1, keepdims=True)
    z = jnp.where(dist == rowmin, iota_s[...], float(K))
    tok = jnp.min(z, axis=1, keepdims=True).astype(jnp.int32)
    tok_ref[...] = jnp.reshape(tok, (BLK // 128, 128))

    colmin_s[...] = jnp.minimum(colmin_s[...], jnp.min(dist, axis=0, keepdims=True))
    zcol_s[...] = jnp.minimum(zcol_s[...], jnp.min(z, axis=0, keepdims=True))
    rowsum_s[0, 0] = rowsum_s[0, 0] + jnp.sum(rowmin)

    @pl.when(i == HGRID - 1)
    def _fin():
        cm_out[...] = colmin_s[...]
        zc_out[...] = zcol_s[...]
        if is_final:
            ent = jnp.sum(jnp.where(zcol_s[...] < float(K), 0.0, colmin_s[...]))
            mean_sq = rowsum_s[0, 0] / (N * D)
            total = _LOSS_W * mean_sq + _ENTROPY_W * ent / K
            rs_out[...] = jnp.full((1, 1), total, jnp.float32)
        else:
            rs_out[...] = jnp.full((1, 1), rowsum_s[0, 0], jnp.float32)


def _tc_half(xh, cb, cm, zc, rs, is_final, interpret=False):
    return pl.pallas_call(
        functools.partial(_tc_body, is_final),
        grid=(HGRID,),
        in_specs=[
            pl.BlockSpec((1, 32, 32, D), lambda i: (i, 0, 0, 0)),
            pl.BlockSpec((K, D), lambda i: (0, 0)),
            pl.BlockSpec((1, K), lambda i: (0, 0)),
            pl.BlockSpec((1, K), lambda i: (0, 0)),
            pl.BlockSpec((1, 1), lambda i: (0, 0)),
        ],
        out_specs=[
            pl.BlockSpec((BLK // 128, 128), lambda i: (i, 0)),
            pl.BlockSpec((1, K), lambda i: (0, 0)),
            pl.BlockSpec((1, K), lambda i: (0, 0)),
            pl.BlockSpec((1, 1), lambda i: (0, 0)),
        ],
        out_shape=[
            jax.ShapeDtypeStruct((NH // 128, 128), jnp.int32),
            jax.ShapeDtypeStruct((1, K), jnp.float32),
            jax.ShapeDtypeStruct((1, K), jnp.float32),
            jax.ShapeDtypeStruct((1, 1), jnp.float32),
        ],
        scratch_shapes=[
            pltpu.VMEM((D, K), jnp.bfloat16),
            pltpu.VMEM((1, K), jnp.float32),
            pltpu.VMEM((1, K), jnp.float32),
            pltpu.VMEM((1, K), jnp.float32),
            pltpu.VMEM((1, K), jnp.float32),
            pltpu.SMEM((1, 1), jnp.float32),
        ],
        interpret=interpret,
    )(xh, cb, cm, zc, rs)


_SC_NC = 2
_SC_NS = 16
_NW = _SC_NC * _SC_NS
_ROWS_W = NH // _NW
_IDX_CH = 128


def _sc_gather(tok2d, codebook):
    mesh = plsc.VectorSubcoreMesh(core_axis_name="c", subcore_axis_name="s")

    @functools.partial(
        pl.kernel,
        mesh=mesh,
        compiler_params=pltpu.CompilerParams(use_tc_tiling_on_sc=False),
        out_type=jax.ShapeDtypeStruct((NH, D), jnp.float32),
        scratch_types=[
            pltpu.VMEM((1, _IDX_CH), jnp.int32),
            pltpu.VMEM((_ROWS_W, D), jnp.float32),
            pltpu.SemaphoreType.DMA,
        ],
    )
    def k(tok_hbm, cb_hbm, out_hbm, idx_v, rows_v, sem):
        wid = lax.axis_index("s") * _SC_NC + lax.axis_index("c")
        pltpu.sync_copy(tok_hbm.at[pl.ds(wid, 1)], idx_v)
        pltpu.async_copy(cb_hbm.at[idx_v.at[0]], rows_v, sem).wait()
        pltpu.sync_copy(rows_v, out_hbm.at[pl.ds(wid * _ROWS_W, _ROWS_W)])

    return k(tok2d, codebook)


def kernel(inputs, codebook):
    cm0 = jnp.full((1, K), jnp.inf, jnp.float32)
    zc0 = jnp.full((1, K), float(K), jnp.float32)
    rs0 = jnp.zeros((1, 1), jnp.float32)
    tokA, cmA, zcA, rsA = _tc_half(inputs[:HALF_B], codebook, cm0, zc0, rs0,
                                   is_final=False)
    embA = _sc_gather(tokA, codebook)
    tokB, _, _, loss = _tc_half(inputs[HALF_B:], codebook, cmA, zcA, rsA,
                                is_final=True)
    embB = _sc_gather(tokB, codebook)
    out = jnp.concatenate([embA, embB], axis=0)
    return out.reshape(inputs.shape), loss[0, 0]


# speedup vs baseline: 1.3222x; 1.3222x over previous
import functools

import jax
import jax.numpy as jnp
from jax import lax
from jax.experimental import pallas as pl
from jax.experimental.pallas import tpu as pltpu
from jax.experimental.pallas import tpu_sc as plsc

N = 8192
D = 64
K = 1024
BLK = 2048
GRID = N // BLK

_LOSS_W = 1.25
_ENTROPY_W = 0.1


def _tc_body(x_ref, cb_ref, tok_ref, loss_ref, cbt_s, c2_s, colmin_s, zcol_s,
             iota_s, rowsum_s):
    i = pl.program_id(0)

    @pl.when(i == 0)
    def _init():
        cbt = cb_ref[...].T
        cbt_s[...] = cbt.astype(jnp.bfloat16)
        c2_s[...] = jnp.sum(cbt * cbt, axis=0, keepdims=True)
        colmin_s[...] = jnp.full((1, K), jnp.inf, jnp.float32)
        zcol_s[...] = jnp.full((1, K), float(K), jnp.float32)
        iota_s[...] = lax.broadcasted_iota(jnp.int32, (1, K), 1).astype(jnp.float32)
        rowsum_s[0, 0] = 0.0

    x = x_ref[...].reshape(BLK, D)
    v2 = jnp.sum(x * x, axis=1, keepdims=True)
    cross2 = jnp.dot((x + x).astype(jnp.bfloat16), cbt_s[...],
                     preferred_element_type=jnp.float32)
    dist = (v2 - cross2) + c2_s[...]

    rowmin = jnp.min(dist, axis=1, keepdims=True)
    z = jnp.where(dist == rowmin, iota_s[...], float(K))
    tok = jnp.min(z, axis=1, keepdims=True).astype(jnp.int32)
    tok_ref[...] = jnp.reshape(tok, (BLK // 128, 128))

    colmin_s[...] = jnp.minimum(colmin_s[...], jnp.min(dist, axis=0, keepdims=True))
    zcol_s[...] = jnp.minimum(zcol_s[...], jnp.min(z, axis=0, keepdims=True))
    rowsum_s[0, 0] = rowsum_s[0, 0] + jnp.sum(rowmin)

    @pl.when(i == GRID - 1)
    def _fin():
        ent = jnp.sum(jnp.where(zcol_s[...] < float(K), 0.0, colmin_s[...]))
        mean_sq = rowsum_s[0, 0] / (N * D)
        total = _LOSS_W * mean_sq + _ENTROPY_W * ent / K
        loss_ref[...] = jnp.full((1, 1), total, jnp.float32)


def _tc_call(inputs4d, cb, interpret=False):
    return pl.pallas_call(
        _tc_body,
        grid=(GRID,),
        in_specs=[
            pl.BlockSpec((2, 32, 32, D), lambda i: (i, 0, 0, 0)),
            pl.BlockSpec((K, D), lambda i: (0, 0)),
        ],
        out_specs=[
            pl.BlockSpec((BLK // 128, 128), lambda i: (i, 0)),
            pl.BlockSpec((1, 1), lambda i: (0, 0)),
        ],
        out_shape=[
            jax.ShapeDtypeStruct((N // 128, 128), jnp.int32),
            jax.ShapeDtypeStruct((1, 1), jnp.float32),
        ],
        scratch_shapes=[
            pltpu.VMEM((D, K), jnp.bfloat16),
            pltpu.VMEM((1, K), jnp.float32),
            pltpu.VMEM((1, K), jnp.float32),
            pltpu.VMEM((1, K), jnp.float32),
            pltpu.VMEM((1, K), jnp.float32),
            pltpu.SMEM((1, 1), jnp.float32),
        ],
        interpret=interpret,
    )(inputs4d, cb)


_SC_NC = 2
_SC_NS = 16
_NW = _SC_NC * _SC_NS
_ROWS_W = N // _NW
_IDX_CH = 128
_CH_W = _ROWS_W // _IDX_CH


def _sc_gather(tok2d, codebook):
    mesh = plsc.VectorSubcoreMesh(core_axis_name="c", subcore_axis_name="s")

    @functools.partial(
        pl.kernel,
        mesh=mesh,
        compiler_params=pltpu.CompilerParams(use_tc_tiling_on_sc=False),
        out_type=jax.ShapeDtypeStruct((N, D), jnp.float32),
        scratch_types=[
            pltpu.VMEM((_CH_W, _IDX_CH), jnp.int32),
            pltpu.VMEM((_ROWS_W, D), jnp.float32),
            pltpu.SemaphoreType.DMA,
        ],
    )
    def k(tok_hbm, cb_hbm, out_hbm, idx_v, rows_v, sem):
        wid = lax.axis_index("s") * _SC_NC + lax.axis_index("c")
        pltpu.sync_copy(tok_hbm.at[pl.ds(wid * _CH_W, _CH_W)], idx_v)
        cp0 = pltpu.async_copy(cb_hbm.at[idx_v.at[0]],
                               rows_v.at[pl.ds(0, _IDX_CH)], sem)
        cp1 = pltpu.async_copy(cb_hbm.at[idx_v.at[1]],
                               rows_v.at[pl.ds(_IDX_CH, _IDX_CH)], sem)
        cp0.wait()
        cp1.wait()
        pltpu.sync_copy(rows_v, out_hbm.at[pl.ds(wid * _ROWS_W, _ROWS_W)])

    return k(tok2d, codebook)


def kernel(inputs, codebook):
    tok, loss = _tc_call(inputs, codebook)
    emb = _sc_gather(tok, codebook)
    return emb.reshape(inputs.shape), loss[0, 0]


# speedup vs baseline: 1.3255x; 1.0025x over previous
import functools

import jax
import jax.numpy as jnp
from jax import lax
from jax.experimental import pallas as pl
from jax.experimental.pallas import tpu as pltpu
from jax.experimental.pallas import tpu_sc as plsc

N = 8192
D = 64
K = 1024
BLK = 4096
GRID = N // BLK

_LOSS_W = 1.25
_ENTROPY_W = 0.1


def _tc_body(x_ref, cb_ref, tok_ref, loss_ref, cbt_s, c2_s, colmin_s, zcol_s,
             iota_s, rowsum_s):
    i = pl.program_id(0)

    @pl.when(i == 0)
    def _init():
        cbt = cb_ref[...].T
        cbt_s[...] = cbt.astype(jnp.bfloat16)
        c2_s[...] = jnp.sum(cbt * cbt, axis=0, keepdims=True)
        colmin_s[...] = jnp.full((1, K), jnp.inf, jnp.float32)
        zcol_s[...] = jnp.full((1, K), float(K), jnp.float32)
        iota_s[...] = lax.broadcasted_iota(jnp.int32, (1, K), 1).astype(jnp.float32)
        rowsum_s[0, 0] = 0.0

    x = x_ref[...].reshape(BLK, D)
    v2 = jnp.sum(x * x, axis=1, keepdims=True)
    cross2 = jnp.dot((x + x).astype(jnp.bfloat16), cbt_s[...],
                     preferred_element_type=jnp.float32)
    dist = (v2 - cross2) + c2_s[...]

    rowmin = jnp.min(dist, axis=1, keepdims=True)
    z = jnp.where(dist == rowmin, iota_s[...], float(K))
    tok = jnp.min(z, axis=1, keepdims=True).astype(jnp.int32)
    tok_ref[...] = jnp.reshape(tok, (BLK // 128, 128))

    colmin_s[...] = jnp.minimum(colmin_s[...], jnp.min(dist, axis=0, keepdims=True))
    zcol_s[...] = jnp.minimum(zcol_s[...], jnp.min(z, axis=0, keepdims=True))
    rowsum_s[0, 0] = rowsum_s[0, 0] + jnp.sum(rowmin)

    @pl.when(i == GRID - 1)
    def _fin():
        ent = jnp.sum(jnp.where(zcol_s[...] < float(K), 0.0, colmin_s[...]))
        mean_sq = rowsum_s[0, 0] / (N * D)
        total = _LOSS_W * mean_sq + _ENTROPY_W * ent / K
        loss_ref[...] = jnp.full((1, 1), total, jnp.float32)


def _tc_call(inputs4d, cb, interpret=False):
    return pl.pallas_call(
        _tc_body,
        grid=(GRID,),
        in_specs=[
            pl.BlockSpec((4, 32, 32, D), lambda i: (i, 0, 0, 0)),
            pl.BlockSpec((K, D), lambda i: (0, 0)),
        ],
        out_specs=[
            pl.BlockSpec((BLK // 128, 128), lambda i: (i, 0)),
            pl.BlockSpec((1, 1), lambda i: (0, 0)),
        ],
        out_shape=[
            jax.ShapeDtypeStruct((N // 128, 128), jnp.int32),
            jax.ShapeDtypeStruct((1, 1), jnp.float32),
        ],
        scratch_shapes=[
            pltpu.VMEM((D, K), jnp.bfloat16),
            pltpu.VMEM((1, K), jnp.float32),
            pltpu.VMEM((1, K), jnp.float32),
            pltpu.VMEM((1, K), jnp.float32),
            pltpu.VMEM((1, K), jnp.float32),
            pltpu.SMEM((1, 1), jnp.float32),
        ],
        interpret=interpret,
    )(inputs4d, cb)


_SC_NC = 2
_SC_NS = 16
_NW = _SC_NC * _SC_NS
_ROWS_W = N // _NW
_IDX_CH = 128
_CH_W = _ROWS_W // _IDX_CH


def _sc_gather(tok2d, codebook):
    mesh = plsc.VectorSubcoreMesh(core_axis_name="c", subcore_axis_name="s")

    @functools.partial(
        pl.kernel,
        mesh=mesh,
        compiler_params=pltpu.CompilerParams(use_tc_tiling_on_sc=False),
        out_type=jax.ShapeDtypeStruct((N, D), jnp.float32),
        scratch_types=[
            pltpu.VMEM((_CH_W, _IDX_CH), jnp.int32),
            pltpu.VMEM((_ROWS_W, D), jnp.float32),
            pltpu.SemaphoreType.DMA,
        ],
    )
    def k(tok_hbm, cb_hbm, out_hbm, idx_v, rows_v, sem):
        wid = lax.axis_index("s") * _SC_NC + lax.axis_index("c")
        pltpu.sync_copy(tok_hbm.at[pl.ds(wid * _CH_W, _CH_W)], idx_v)
        cp0 = pltpu.async_copy(cb_hbm.at[idx_v.at[0]],
                               rows_v.at[pl.ds(0, _IDX_CH)], sem)
        cp1 = pltpu.async_copy(cb_hbm.at[idx_v.at[1]],
                               rows_v.at[pl.ds(_IDX_CH, _IDX_CH)], sem)
        cp0.wait()
        cp1.wait()
        pltpu.sync_copy(rows_v, out_hbm.at[pl.ds(wid * _ROWS_W, _ROWS_W)])

    return k(tok2d, codebook)


def kernel(inputs, codebook):
    tok, loss = _tc_call(inputs, codebook)
    emb = _sc_gather(tok, codebook)
    return emb.reshape(inputs.shape), loss[0, 0]
